# Initial kernel scaffold; baseline (speedup 1.0000x reference)
import jax
import jax.numpy as jnp
from jax.experimental import pallas as pl


def kernel(user_table, item_table, g_idx, g_val, s1_idx, s1_val, s2_idx, s2_val, user_id, item_id, neg_item_id):
    raise NotImplementedError("write your pallas kernel here")


# speedup vs baseline: 5.9054x; 5.9054x over previous
import jax
import jax.numpy as jnp
from jax import lax
from jax.experimental import pallas as pl
from jax.experimental.pallas import tpu as pltpu
from jax.experimental.pallas import tpu_sc as plsc

U = 50000
N = 100000
D = 64
CW = 16
NQ = D // CW
B = 4096
TAU = 0.2
LMBD_REG = 1e-4
LMBD_SSL = 0.1

NC, NS, L = 2, 16, 16
NW = NC * NS
EM = 128
RW = 8
HALF = U
CHP = 50176
PT = 3136
PT_LAST = HALF - PT
NPAD = 102400

_mesh = plsc.VectorSubcoreMesh(core_axis_name="c", subcore_axis_name="s")
_sc_params = pltpu.CompilerParams(use_tc_tiling_on_sc=False)

_DRAIN_BLKS = ((0, 1024), (1024, 1024), (2048, 1024), (3072, PT - 3072))


def _zero_fill(ref, nvec):
    def body(i, _):
        ref[pl.ds(i * L, L)] = jnp.zeros((L,), jnp.float32)
        return 0
    lax.fori_loop(0, nvec, body, 0)


def _scale_rows(dst_v, sc_v, off, bs, fn):
    def gloop(g, _):
        chunk = sc_v[pl.ds(off + g * L, L)]
        for k in range(L):
            r = g * L + k
            dst_v[r, :] = fn(r, jnp.full((L,), chunk[k]))
        return 0
    lax.fori_loop(0, bs // L, gloop, 0)


def _make_deg(h_g, h_s):
    hr_g, hr_s = h_g // EM, h_s // EM

    def body(dg, ds1, ds2, og, os1, os2, idx_v, ones_v, zb_v, st_v, deg_sh):
        c = lax.axis_index("c")
        s = lax.axis_index("s")
        base = c * HALF
        startz = s * PT
        startd = jnp.minimum(s * PT, PT_LAST)

        def ob(i, _):
            ones_v[pl.ds(i * L, L)] = jnp.ones((L,), jnp.float32)
            return 0
        lax.fori_loop(0, EM // L, ob, 0)
        _zero_fill(zb_v, PT // L)

        for dst2d, out, hr in ((dg, og, hr_g), (ds1, os1, hr_s), (ds2, os2, hr_s)):
            pltpu.sync_copy(zb_v, deg_sh.at[pl.ds(startz, PT)])
            plsc.subcore_barrier()
            nw = hr // RW
            def wloop(j, _):
                w = j * NS + s
                @pl.when(w < nw)
                def _():
                    r0 = c * hr + w * RW
                    pltpu.sync_copy(dst2d.at[pl.ds(r0, RW)], idx_v)
                    for i in range(RW):
                        for k in range(EM // L):
                            idx_v[i, pl.ds(k * L, L)] = (
                                idx_v[i, pl.ds(k * L, L)] - base)
                        pltpu.sync_copy(ones_v, deg_sh.at[idx_v.at[i]], add=True)
                return 0
            lax.fori_loop(0, pl.cdiv(nw, NS), wloop, 0)
            plsc.subcore_barrier()
            pltpu.sync_copy(deg_sh.at[pl.ds(startd, PT)], st_v)
            pltpu.sync_copy(st_v, out.at[pl.ds(base + startd, PT)])
            plsc.subcore_barrier()

    shp = jax.ShapeDtypeStruct((N,), jnp.float32)
    return pl.kernel(
        body,
        out_type=(shp, shp, shp),
        mesh=_mesh,
        compiler_params=_sc_params,
        scratch_types=[
            pltpu.VMEM((RW, EM), jnp.int32),
            pltpu.VMEM((EM,), jnp.float32),
            pltpu.VMEM((PT,), jnp.float32),
            pltpu.VMEM((PT,), jnp.float32),
            pltpu.VMEM_SHARED((CHP,), jnp.float32),
        ],
    )


def _tc_dinv(deg_g, deg_a, deg_b):
    def body(dg, da, db, ig, ia, ib, qg, qa, qb):
        for dref, iref, qref in ((dg, ig, qg), (da, ia, qa), (db, ib, qb)):
            m = jnp.maximum(dref[...], 1.0)
            dinv = lax.rsqrt(m)
            iref[...] = dinv
            qref[...] = dinv * dinv

    sp = jax.ShapeDtypeStruct((NPAD // 128, 128), jnp.float32)
    pad = lambda d: jnp.pad(d, (0, NPAD - N)).reshape(NPAD // 128, 128)
    outs = pl.pallas_call(
        body,
        out_shape=[sp] * 6,
    )(pad(deg_g), pad(deg_a), pad(deg_b))
    return [o.reshape(NPAD)[:N] for o in outs]


def _scale_body(e0, ivg, iva, ivb, og, oa, ob, e_v, o_v, dg_v, da_v, db_v):
    c = lax.axis_index("c")
    s = lax.axis_index("s")
    wid = s * NC + c
    start = jnp.minimum(wid * PT, N - PT)

    pltpu.sync_copy(ivg.at[pl.ds(start, PT)], dg_v)
    pltpu.sync_copy(iva.at[pl.ds(start, PT)], da_v)
    pltpu.sync_copy(ivb.at[pl.ds(start, PT)], db_v)

    for off, bs in _DRAIN_BLKS:
        pltpu.sync_copy(e0.at[pl.ds(start + off, bs)], e_v.at[pl.ds(0, bs)])
        for dv_v, out in ((dg_v, og), (da_v, oa), (db_v, ob)):
            def qloop(q, _):
                _scale_rows(o_v, dv_v, off, bs,
                            lambda r, sc: e_v[r, pl.ds(q * CW, CW)] * sc)
                pltpu.sync_copy(o_v.at[pl.ds(0, bs)],
                                out.at[pl.ds(q * N + start + off, bs)])
                return 0
            lax.fori_loop(0, NQ, qloop, 0)


def _sc_scale(e0, ivg, iva, ivb):
    sq = jax.ShapeDtypeStruct((NQ * N, CW), jnp.float32)
    return pl.kernel(
        _scale_body,
        out_type=(sq, sq, sq),
        mesh=_mesh,
        compiler_params=_sc_params,
        scratch_types=[
            pltpu.VMEM((1024, D), jnp.float32),
            pltpu.VMEM((1024, CW), jnp.float32),
            pltpu.VMEM((PT,), jnp.float32),
            pltpu.VMEM((PT,), jnp.float32),
            pltpu.VMEM((PT,), jnp.float32),
        ],
    )(e0, ivg, iva, ivb)


def _make_spmm(h, scaled):
    hr = h // EM

    def body(*refs):
        if scaled:
            (dst2d, col2d, y, scv, u, dst_v, col_v, cq_v, rows_v, zb_v, st_v,
             sc_v, chunk_sh, sem) = refs
        else:
            (dst2d, col2d, y, u, dst_v, col_v, cq_v, rows_v, zb_v, st_v,
             sc_v, chunk_sh, sem) = refs
            scv = None
        c = lax.axis_index("c")
        s = lax.axis_index("s")
        base = c * HALF
        startz = s * PT
        startd = jnp.minimum(s * PT, PT_LAST)
        nw = hr // RW

        def zb(i, _):
            zb_v[i, :] = jnp.zeros((L,), jnp.float32)
            return 0
        lax.fori_loop(0, 1024, zb, 0)
        if scaled:
            pltpu.sync_copy(scv.at[pl.ds(base + startd, PT)], sc_v)

        def qpass(q, _):
            qoff = q * N
            for zo, zs in _DRAIN_BLKS:
                pltpu.sync_copy(zb_v.at[pl.ds(0, zs)],
                                chunk_sh.at[pl.ds(startz + zo, zs)])
            plsc.subcore_barrier()

            def wloop(j, _):
                w = j * NS + s
                @pl.when(w < nw)
                def _():
                    r0 = c * hr + w * RW
                    pltpu.sync_copy(dst2d.at[pl.ds(r0, RW)], dst_v)
                    pltpu.sync_copy(col2d.at[pl.ds(r0, RW)], col_v)
                    for i in range(RW):
                        for k in range(EM // L):
                            dst_v[i, pl.ds(k * L, L)] = (
                                dst_v[i, pl.ds(k * L, L)] - base)
                            cq_v[i, pl.ds(k * L, L)] = (
                                col_v[i, pl.ds(k * L, L)] + qoff)
                    cps = [pltpu.async_copy(y.at[cq_v.at[i]], rows_v.at[i], sem)
                           for i in range(RW)]
                    for cp in cps:
                        cp.wait()
                    for i in range(RW):
                        pltpu.sync_copy(rows_v.at[i],
                                        chunk_sh.at[dst_v.at[i]], add=True)
                return 0
            lax.fori_loop(0, pl.cdiv(nw, NS), wloop, 0)
            plsc.subcore_barrier()

            for off, bs in _DRAIN_BLKS:
                pltpu.sync_copy(chunk_sh.at[pl.ds(startd + off, bs)],
                                st_v.at[pl.ds(0, bs)])
                if scaled:
                    _scale_rows(st_v, sc_v, off, bs,
                                lambda r, sc: st_v[r, :] * sc)
                pltpu.sync_copy(st_v.at[pl.ds(0, bs)],
                                u.at[pl.ds(qoff + base + startd + off, bs)])
            plsc.subcore_barrier()
            return 0
        lax.fori_loop(0, NQ, qpass, 0)

    sq = jax.ShapeDtypeStruct((NQ * N, CW), jnp.float32)
    return pl.kernel(
        body,
        out_type=sq,
        mesh=_mesh,
        compiler_params=_sc_params,
        scratch_types=[
            pltpu.VMEM((RW, EM), jnp.int32),
            pltpu.VMEM((RW, EM), jnp.int32),
            pltpu.VMEM((RW, EM), jnp.int32),
            pltpu.VMEM((RW, EM, CW), jnp.float32),
            pltpu.VMEM((1024, CW), jnp.float32),
            pltpu.VMEM((1024, CW), jnp.float32),
            pltpu.VMEM((PT,), jnp.float32),
            pltpu.VMEM_SHARED((CHP, CW), jnp.float32),
            pltpu.SemaphoreType.DMA,
        ],
    )


BW = B // NW


def _gather_body(uid, iid, nid, utab, itab, ivg, iva, ivb, y1g, y1a, y1b,
                 u2g, u2a, u2b,
                 ue0, pe0, ne0, w1g_u, w1g_i, w1g_n, w1a_u, w1b_u, w1a_i,
                 w1b_i, w2g_u, w2g_i, w2g_n, w2a_u, w2b_u, w2a_i, w2b_i,
                 dv_g_u, dv_g_i, dv_g_n, dv_a_u, dv_b_u, dv_a_i, dv_b_i,
                 uid_v, iid_v, nid_v, inode_v, nnode_v, qidx_v,
                 q_v, w_v, r64_v, dv_v, sem):
    c = lax.axis_index("c")
    s = lax.axis_index("s")
    wid = s * NC + c
    o = wid * BW

    pltpu.sync_copy(uid.at[pl.ds(o, BW)], uid_v)
    pltpu.sync_copy(iid.at[pl.ds(o, BW)], iid_v)
    pltpu.sync_copy(nid.at[pl.ds(o, BW)], nid_v)
    def off(i, _):
        inode_v[pl.ds(i * L, L)] = iid_v[pl.ds(i * L, L)] + U
        nnode_v[pl.ds(i * L, L)] = nid_v[pl.ds(i * L, L)] + U
        return 0
    lax.fori_loop(0, BW // L, off, 0)

    for tab, idx_v, out in ((utab, uid_v, ue0), (itab, iid_v, pe0),
                            (itab, nid_v, ne0)):
        pltpu.async_copy(tab.at[idx_v], r64_v, sem).wait()
        pltpu.sync_copy(r64_v, out.at[pl.ds(o, BW)])

    combos = ((uid_v, y1g, w1g_u), (inode_v, y1g, w1g_i), (nnode_v, y1g, w1g_n),
              (uid_v, y1a, w1a_u), (uid_v, y1b, w1b_u),
              (inode_v, y1a, w1a_i), (inode_v, y1b, w1b_i),
              (uid_v, u2g, w2g_u), (inode_v, u2g, w2g_i), (nnode_v, u2g, w2g_n),
              (uid_v, u2a, w2a_u), (uid_v, u2b, w2b_u),
              (inode_v, u2a, w2a_i), (inode_v, u2b, w2b_i))
    for idx_v, src, out in combos:
        for q in range(NQ):
            def qi(i, _):
                qidx_v[pl.ds(i * L, L)] = idx_v[pl.ds(i * L, L)] + q * N
                return 0
            lax.fori_loop(0, BW // L, qi, 0)
            pltpu.async_copy(src.at[qidx_v], q_v, sem).wait()
            def cpy(i, _):
                w_v[i, pl.ds(q * CW, CW)] = q_v[i, :]
                return 0
            lax.fori_loop(0, BW, cpy, 0)
        pltpu.sync_copy(w_v, out.at[pl.ds(o, BW)])

    for iv, idx_v, out in ((ivg, uid_v, dv_g_u), (ivg, inode_v, dv_g_i),
                           (ivg, nnode_v, dv_g_n), (iva, uid_v, dv_a_u),
                           (ivb, uid_v, dv_b_u), (iva, inode_v, dv_a_i),
                           (ivb, inode_v, dv_b_i)):
        pltpu.async_copy(iv.at[idx_v], dv_v, sem).wait()
        pltpu.sync_copy(dv_v, out.at[pl.ds(o, BW)])


def _sc_gather(*args):
    sr = jax.ShapeDtypeStruct((B, D), jnp.float32)
    sv = jax.ShapeDtypeStruct((B,), jnp.float32)
    return pl.kernel(
        _gather_body,
        out_type=(sr,) * 17 + (sv,) * 7,
        mesh=_mesh,
        compiler_params=_sc_params,
        scratch_types=[
            pltpu.VMEM((BW,), jnp.int32),
            pltpu.VMEM((BW,), jnp.int32),
            pltpu.VMEM((BW,), jnp.int32),
            pltpu.VMEM((BW,), jnp.int32),
            pltpu.VMEM((BW,), jnp.int32),
            pltpu.VMEM((BW,), jnp.int32),
            pltpu.VMEM((BW, CW), jnp.float32),
            pltpu.VMEM((BW, D), jnp.float32),
            pltpu.VMEM((BW, D), jnp.float32),
            pltpu.VMEM((BW,), jnp.float32),
            pltpu.SemaphoreType.DMA,
        ],
    )(*args)


def _light(e0r, w1, w2, dv):
    d = dv[...][:, None]
    return (e0r[...] + w1[...] / d + d * w2[...]) * jnp.float32(1.0 / 3.0)


def _bpr_body(ue0, pe0, ne0, w1u, w2u, dvu, w1i, w2i, dvi, w1n, w2n, dvn, out):
    ue = _light(ue0, w1u, w2u, dvu)
    pe = _light(pe0, w1i, w2i, dvi)
    ne = _light(ne0, w1n, w2n, dvn)
    x = jnp.sum(ue * pe, axis=1) - jnp.sum(ue * ne, axis=1)
    logsig = jnp.minimum(x, 0.0) - jnp.log(1.0 + jnp.exp(-jnp.abs(x)))
    bpr = -jnp.mean(logsig)
    reg = (LMBD_REG * 0.5 / B) * (jnp.sum(ue0[...] ** 2) + jnp.sum(pe0[...] ** 2)
                                  + jnp.sum(ne0[...] ** 2))
    out[0, 0] = bpr + reg * LMBD_REG


def _nce_body(e0, w1a, w2a, dva, w1b, w2b, dvb, out, z2_s):
    z1 = _light(e0, w1a, w2a, dva)
    z2 = _light(e0, w1b, w2b, dvb)
    n1 = jnp.sqrt(jnp.sum(z1 * z1, axis=1, keepdims=True)) + 1e-12
    n2 = jnp.sqrt(jnp.sum(z2 * z2, axis=1, keepdims=True)) + 1e-12
    z1n = z1 / n1
    z2_s[...] = z2 / n2
    posd = jnp.sum(z1n * z2_s[...], axis=1) * (1.0 / TAU)

    def jstep(j, acc):
        tile = z2_s[pl.ds(j * 128, 128), :]
        t = lax.dot_general(z1n, tile, (((1,), (1,)), ((), ())),
                            preferred_element_type=jnp.float32)
        return acc + jnp.sum(jnp.exp(t * (1.0 / TAU)), axis=1)

    acc = lax.fori_loop(0, B // 128, jstep, jnp.zeros((B,), jnp.float32))
    out[0, 0] = jnp.mean(jnp.log(acc) - posd)


def _scalar_call(body, n_in, scratch=()):
    def run(*args):
        return pl.pallas_call(
            body,
            out_specs=pl.BlockSpec(memory_space=pltpu.SMEM),
            out_shape=jax.ShapeDtypeStruct((1, 1), jnp.float32),
            scratch_shapes=list(scratch),
        )(*args)
    return run


def kernel(user_table, item_table, g_idx, g_val, s1_idx, s1_val, s2_idx,
           s2_val, user_id, item_id, neg_item_id):
    del g_val, s1_val, s2_val
    e0 = jnp.concatenate([user_table, item_table], axis=0)

    wnd = RW * EM

    def pad_split(idx):
        h = idx.shape[1] // 2
        hp = ((h + wnd - 1) // wnd) * wnd
        pad = hp - h
        dstu = jnp.concatenate([idx[0, :h], jnp.full((pad,), HALF, jnp.int32)])
        dsti = jnp.concatenate([idx[0, h:], jnp.full((pad,), N, jnp.int32)])
        colu = jnp.concatenate([idx[1, :h], jnp.zeros((pad,), jnp.int32)])
        coli = jnp.concatenate([idx[1, h:], jnp.zeros((pad,), jnp.int32)])
        dst2 = jnp.concatenate([dstu, dsti]).reshape(-1, EM)
        col2 = jnp.concatenate([colu, coli]).reshape(-1, EM)
        return dst2, col2, hp

    dst_g, col_g, hp_g = pad_split(g_idx)
    dst_a, col_a, hp_s = pad_split(s1_idx)
    dst_b, col_b, _ = pad_split(s2_idx)

    deg_g, deg_a, deg_b = _make_deg(hp_g, hp_s)(dst_g, dst_a, dst_b)
    ivg, iva, ivb, qvg, qva, qvb = _tc_dinv(deg_g, deg_a, deg_b)

    y0g, y0a, y0b = _sc_scale(e0, ivg, iva, ivb)

    spmm1_g = _make_spmm(hp_g, scaled=True)
    spmm1_s = _make_spmm(hp_s, scaled=True)
    y1g = spmm1_g(dst_g, col_g, y0g, qvg)
    y1a = spmm1_s(dst_a, col_a, y0a, qva)
    y1b = spmm1_s(dst_b, col_b, y0b, qvb)

    spmm2_g = _make_spmm(hp_g, scaled=False)
    spmm2_s = _make_spmm(hp_s, scaled=False)
    u2g = spmm2_g(dst_g, col_g, y1g)
    u2a = spmm2_s(dst_a, col_a, y1a)
    u2b = spmm2_s(dst_b, col_b, y1b)

    (ue0, pe0, ne0, w1g_u, w1g_i, w1g_n, w1a_u, w1b_u, w1a_i, w1b_i,
     w2g_u, w2g_i, w2g_n, w2a_u, w2b_u, w2a_i, w2b_i,
     dv_g_u, dv_g_i, dv_g_n, dv_a_u, dv_b_u, dv_a_i, dv_b_i) = _sc_gather(
        user_id, item_id, neg_item_id, user_table, item_table,
        ivg, iva, ivb, y1g, y1a, y1b, u2g, u2a, u2b,
    )

    bpr_reg = _scalar_call(_bpr_body, 12)(
        ue0, pe0, ne0, w1g_u, w2g_u, dv_g_u, w1g_i, w2g_i, dv_g_i,
        w1g_n, w2g_n, dv_g_n)
    nce = _scalar_call(_nce_body, 7, scratch=[pltpu.VMEM((B, D), jnp.float32)])
    ssl_u = nce(ue0, w1a_u, w2a_u, dv_a_u, w1b_u, w2b_u, dv_b_u)
    ssl_i = nce(pe0, w1a_i, w2a_i, dv_a_i, w1b_i, w2b_i, dv_b_i)

    total = bpr_reg[0, 0] + (ssl_u[0, 0] + ssl_i[0, 0]) * LMBD_SSL
    return total


# speedup vs baseline: 6.5557x; 1.1101x over previous
import jax
import jax.numpy as jnp
from jax import lax
from jax.experimental import pallas as pl
from jax.experimental.pallas import tpu as pltpu
from jax.experimental.pallas import tpu_sc as plsc

U = 50000
N = 100000
D = 64
CW = 16
NQ = D // CW
B = 4096
TAU = 0.2
LMBD_REG = 1e-4
LMBD_SSL = 0.1

NC, NS, L = 2, 16, 16
NW = NC * NS
EM = 128
RW = 8
HALF = U
CHP = 50176
PT = 3136
PT_LAST = HALF - PT
NPAD = 102400

_mesh = plsc.VectorSubcoreMesh(core_axis_name="c", subcore_axis_name="s")
_sc_params = pltpu.CompilerParams(use_tc_tiling_on_sc=False)

_DRAIN_BLKS = ((0, 1024), (1024, 1024), (2048, 1024), (3072, PT - 3072))


def _zero_fill(ref, nvec):
    def body(i, _):
        ref[pl.ds(i * L, L)] = jnp.zeros((L,), jnp.float32)
        return 0
    lax.fori_loop(0, nvec, body, 0)


def _scale_rows(dst_v, sc_v, off, bs, fn):
    def gloop(g, _):
        chunk = sc_v[pl.ds(off + g * L, L)]
        for k in range(L):
            r = g * L + k
            dst_v[r, :] = fn(r, jnp.full((L,), chunk[k]))
        return 0
    lax.fori_loop(0, bs // L, gloop, 0)


def _make_deg(h_g, h_s):
    hr_g, hr_s = h_g // EM, h_s // EM

    def body(dg, ds1, ds2, og, os1, os2, idx_v, ones_v, zb_v, st_v, deg_sh,
             dsem):
        c = lax.axis_index("c")
        s = lax.axis_index("s")
        base = c * HALF
        startz = s * PT
        startd = jnp.minimum(s * PT, PT_LAST)

        def ob(i, _):
            ones_v[pl.ds(i * L, L)] = jnp.ones((L,), jnp.float32)
            return 0
        lax.fori_loop(0, EM // L, ob, 0)
        _zero_fill(zb_v, PT // L)

        for dst2d, out, hr in ((dg, og, hr_g), (ds1, os1, hr_s), (ds2, os2, hr_s)):
            pltpu.sync_copy(zb_v, deg_sh.at[pl.ds(startz, PT)])
            plsc.subcore_barrier()
            nw = hr // RW
            def wloop(j, _):
                w = j * NS + s
                @pl.when(w < nw)
                def _():
                    r0 = c * hr + w * RW
                    pltpu.sync_copy(dst2d.at[pl.ds(r0, RW)], idx_v)
                    for i in range(RW):
                        for k in range(EM // L):
                            idx_v[i, pl.ds(k * L, L)] = (
                                idx_v[i, pl.ds(k * L, L)] - base)
                    sps = [pltpu.async_copy(ones_v, deg_sh.at[idx_v.at[i]],
                                            dsem, add=True)
                           for i in range(RW)]
                    for sp in sps:
                        sp.wait()
                return 0
            lax.fori_loop(0, pl.cdiv(nw, NS), wloop, 0)
            plsc.subcore_barrier()
            pltpu.sync_copy(deg_sh.at[pl.ds(startd, PT)], st_v)
            pltpu.sync_copy(st_v, out.at[pl.ds(base + startd, PT)])
            plsc.subcore_barrier()

    shp = jax.ShapeDtypeStruct((N,), jnp.float32)
    return pl.kernel(
        body,
        out_type=(shp, shp, shp),
        mesh=_mesh,
        compiler_params=_sc_params,
        scratch_types=[
            pltpu.VMEM((RW, EM), jnp.int32),
            pltpu.VMEM((EM,), jnp.float32),
            pltpu.VMEM((PT,), jnp.float32),
            pltpu.VMEM((PT,), jnp.float32),
            pltpu.VMEM_SHARED((CHP,), jnp.float32),
            pltpu.SemaphoreType.DMA,
        ],
    )


def _tc_dinv(deg_g, deg_a, deg_b):
    def body(dg, da, db, ig, ia, ib, qg, qa, qb):
        for dref, iref, qref in ((dg, ig, qg), (da, ia, qa), (db, ib, qb)):
            m = jnp.maximum(dref[...], 1.0)
            dinv = lax.rsqrt(m)
            iref[...] = dinv
            qref[...] = dinv * dinv

    sp = jax.ShapeDtypeStruct((NPAD // 128, 128), jnp.float32)
    pad = lambda d: jnp.pad(d, (0, NPAD - N)).reshape(NPAD // 128, 128)
    outs = pl.pallas_call(
        body,
        out_shape=[sp] * 6,
    )(pad(deg_g), pad(deg_a), pad(deg_b))
    return [o.reshape(NPAD)[:N] for o in outs]


def _scale_body(e0, ivg, iva, ivb, og, oa, ob, e_v, o_v, dg_v, da_v, db_v):
    c = lax.axis_index("c")
    s = lax.axis_index("s")
    wid = s * NC + c
    start = jnp.minimum(wid * PT, N - PT)

    pltpu.sync_copy(ivg.at[pl.ds(start, PT)], dg_v)
    pltpu.sync_copy(iva.at[pl.ds(start, PT)], da_v)
    pltpu.sync_copy(ivb.at[pl.ds(start, PT)], db_v)

    for off, bs in _DRAIN_BLKS:
        pltpu.sync_copy(e0.at[pl.ds(start + off, bs)], e_v.at[pl.ds(0, bs)])
        for dv_v, out in ((dg_v, og), (da_v, oa), (db_v, ob)):
            def qloop(q, _):
                _scale_rows(o_v, dv_v, off, bs,
                            lambda r, sc: e_v[r, pl.ds(q * CW, CW)] * sc)
                pltpu.sync_copy(o_v.at[pl.ds(0, bs)],
                                out.at[pl.ds(q * N + start + off, bs)])
                return 0
            lax.fori_loop(0, NQ, qloop, 0)


def _sc_scale(e0, ivg, iva, ivb):
    sq = jax.ShapeDtypeStruct((NQ * N, CW), jnp.float32)
    return pl.kernel(
        _scale_body,
        out_type=(sq, sq, sq),
        mesh=_mesh,
        compiler_params=_sc_params,
        scratch_types=[
            pltpu.VMEM((1024, D), jnp.float32),
            pltpu.VMEM((1024, CW), jnp.float32),
            pltpu.VMEM((PT,), jnp.float32),
            pltpu.VMEM((PT,), jnp.float32),
            pltpu.VMEM((PT,), jnp.float32),
        ],
    )(e0, ivg, iva, ivb)


def _make_spmm(h, scaled):
    hr = h // EM

    def body(*refs):
        if scaled:
            (dst2d, col2d, y, scv, u, dst_v, col_v, cq_v, rows_v, zb_v, st_v,
             sc_v, chunk_sh, sem) = refs
        else:
            (dst2d, col2d, y, u, dst_v, col_v, cq_v, rows_v, zb_v, st_v,
             sc_v, chunk_sh, sem) = refs
            scv = None
        c = lax.axis_index("c")
        s = lax.axis_index("s")
        base = c * HALF
        startz = s * PT
        startd = jnp.minimum(s * PT, PT_LAST)
        nw = hr // RW

        def zb(i, _):
            zb_v[i, :] = jnp.zeros((L,), jnp.float32)
            return 0
        lax.fori_loop(0, 1024, zb, 0)
        if scaled:
            pltpu.sync_copy(scv.at[pl.ds(base + startd, PT)], sc_v)

        def qpass(q, _):
            qoff = q * N
            for zo, zs in _DRAIN_BLKS:
                pltpu.sync_copy(zb_v.at[pl.ds(0, zs)],
                                chunk_sh.at[pl.ds(startz + zo, zs)])
            plsc.subcore_barrier()

            def wloop(j, _):
                w = j * NS + s
                @pl.when(w < nw)
                def _():
                    r0 = c * hr + w * RW
                    pltpu.sync_copy(dst2d.at[pl.ds(r0, RW)], dst_v)
                    pltpu.sync_copy(col2d.at[pl.ds(r0, RW)], col_v)
                    for i in range(RW):
                        for k in range(EM // L):
                            dst_v[i, pl.ds(k * L, L)] = (
                                dst_v[i, pl.ds(k * L, L)] - base)
                            cq_v[i, pl.ds(k * L, L)] = (
                                col_v[i, pl.ds(k * L, L)] + qoff)
                    cps = [pltpu.async_copy(y.at[cq_v.at[i]], rows_v.at[i], sem)
                           for i in range(RW)]
                    for cp in cps:
                        cp.wait()
                    sps = [pltpu.async_copy(rows_v.at[i],
                                            chunk_sh.at[dst_v.at[i]], sem,
                                            add=True)
                           for i in range(RW)]
                    for sp in sps:
                        sp.wait()
                return 0
            lax.fori_loop(0, pl.cdiv(nw, NS), wloop, 0)
            plsc.subcore_barrier()

            for off, bs in _DRAIN_BLKS:
                pltpu.sync_copy(chunk_sh.at[pl.ds(startd + off, bs)],
                                st_v.at[pl.ds(0, bs)])
                if scaled:
                    _scale_rows(st_v, sc_v, off, bs,
                                lambda r, sc: st_v[r, :] * sc)
                pltpu.sync_copy(st_v.at[pl.ds(0, bs)],
                                u.at[pl.ds(qoff + base + startd + off, bs)])
            plsc.subcore_barrier()
            return 0
        lax.fori_loop(0, NQ, qpass, 0)

    sq = jax.ShapeDtypeStruct((NQ * N, CW), jnp.float32)
    return pl.kernel(
        body,
        out_type=sq,
        mesh=_mesh,
        compiler_params=_sc_params,
        scratch_types=[
            pltpu.VMEM((RW, EM), jnp.int32),
            pltpu.VMEM((RW, EM), jnp.int32),
            pltpu.VMEM((RW, EM), jnp.int32),
            pltpu.VMEM((RW, EM, CW), jnp.float32),
            pltpu.VMEM((1024, CW), jnp.float32),
            pltpu.VMEM((1024, CW), jnp.float32),
            pltpu.VMEM((PT,), jnp.float32),
            pltpu.VMEM_SHARED((CHP, CW), jnp.float32),
            pltpu.SemaphoreType.DMA,
        ],
    )


BW = B // NW


def _gather_body(uid, iid, nid, utab, itab, ivg, iva, ivb, y1g, y1a, y1b,
                 u2g, u2a, u2b,
                 ue0, pe0, ne0, w1g_u, w1g_i, w1g_n, w1a_u, w1b_u, w1a_i,
                 w1b_i, w2g_u, w2g_i, w2g_n, w2a_u, w2b_u, w2a_i, w2b_i,
                 dv_g_u, dv_g_i, dv_g_n, dv_a_u, dv_b_u, dv_a_i, dv_b_i,
                 uid_v, iid_v, nid_v, inode_v, nnode_v, qidx_v,
                 q_v, w_v, r64_v, dv_v, sem):
    c = lax.axis_index("c")
    s = lax.axis_index("s")
    wid = s * NC + c
    o = wid * BW

    pltpu.sync_copy(uid.at[pl.ds(o, BW)], uid_v)
    pltpu.sync_copy(iid.at[pl.ds(o, BW)], iid_v)
    pltpu.sync_copy(nid.at[pl.ds(o, BW)], nid_v)
    def off(i, _):
        inode_v[pl.ds(i * L, L)] = iid_v[pl.ds(i * L, L)] + U
        nnode_v[pl.ds(i * L, L)] = nid_v[pl.ds(i * L, L)] + U
        return 0
    lax.fori_loop(0, BW // L, off, 0)

    for tab, idx_v, out in ((utab, uid_v, ue0), (itab, iid_v, pe0),
                            (itab, nid_v, ne0)):
        pltpu.async_copy(tab.at[idx_v], r64_v, sem).wait()
        pltpu.sync_copy(r64_v, out.at[pl.ds(o, BW)])

    combos = ((uid_v, y1g, w1g_u), (inode_v, y1g, w1g_i), (nnode_v, y1g, w1g_n),
              (uid_v, y1a, w1a_u), (uid_v, y1b, w1b_u),
              (inode_v, y1a, w1a_i), (inode_v, y1b, w1b_i),
              (uid_v, u2g, w2g_u), (inode_v, u2g, w2g_i), (nnode_v, u2g, w2g_n),
              (uid_v, u2a, w2a_u), (uid_v, u2b, w2b_u),
              (inode_v, u2a, w2a_i), (inode_v, u2b, w2b_i))
    for idx_v, src, out in combos:
        for q in range(NQ):
            def qi(i, _):
                qidx_v[pl.ds(i * L, L)] = idx_v[pl.ds(i * L, L)] + q * N
                return 0
            lax.fori_loop(0, BW // L, qi, 0)
            pltpu.async_copy(src.at[qidx_v], q_v, sem).wait()
            def cpy(i, _):
                w_v[i, pl.ds(q * CW, CW)] = q_v[i, :]
                return 0
            lax.fori_loop(0, BW, cpy, 0)
        pltpu.sync_copy(w_v, out.at[pl.ds(o, BW)])

    for iv, idx_v, out in ((ivg, uid_v, dv_g_u), (ivg, inode_v, dv_g_i),
                           (ivg, nnode_v, dv_g_n), (iva, uid_v, dv_a_u),
                           (ivb, uid_v, dv_b_u), (iva, inode_v, dv_a_i),
                           (ivb, inode_v, dv_b_i)):
        pltpu.async_copy(iv.at[idx_v], dv_v, sem).wait()
        pltpu.sync_copy(dv_v, out.at[pl.ds(o, BW)])


def _sc_gather(*args):
    sr = jax.ShapeDtypeStruct((B, D), jnp.float32)
    sv = jax.ShapeDtypeStruct((B,), jnp.float32)
    return pl.kernel(
        _gather_body,
        out_type=(sr,) * 17 + (sv,) * 7,
        mesh=_mesh,
        compiler_params=_sc_params,
        scratch_types=[
            pltpu.VMEM((BW,), jnp.int32),
            pltpu.VMEM((BW,), jnp.int32),
            pltpu.VMEM((BW,), jnp.int32),
            pltpu.VMEM((BW,), jnp.int32),
            pltpu.VMEM((BW,), jnp.int32),
            pltpu.VMEM((BW,), jnp.int32),
            pltpu.VMEM((BW, CW), jnp.float32),
            pltpu.VMEM((BW, D), jnp.float32),
            pltpu.VMEM((BW, D), jnp.float32),
            pltpu.VMEM((BW,), jnp.float32),
            pltpu.SemaphoreType.DMA,
        ],
    )(*args)


def _light(e0r, w1, w2, dv):
    d = dv[...][:, None]
    return (e0r[...] + w1[...] / d + d * w2[...]) * jnp.float32(1.0 / 3.0)


def _bpr_body(ue0, pe0, ne0, w1u, w2u, dvu, w1i, w2i, dvi, w1n, w2n, dvn, out):
    ue = _light(ue0, w1u, w2u, dvu)
    pe = _light(pe0, w1i, w2i, dvi)
    ne = _light(ne0, w1n, w2n, dvn)
    x = jnp.sum(ue * pe, axis=1) - jnp.sum(ue * ne, axis=1)
    logsig = jnp.minimum(x, 0.0) - jnp.log(1.0 + jnp.exp(-jnp.abs(x)))
    bpr = -jnp.mean(logsig)
    reg = (LMBD_REG * 0.5 / B) * (jnp.sum(ue0[...] ** 2) + jnp.sum(pe0[...] ** 2)
                                  + jnp.sum(ne0[...] ** 2))
    out[0, 0] = bpr + reg * LMBD_REG


def _nce_body(e0, w1a, w2a, dva, w1b, w2b, dvb, out, z2_s):
    z1 = _light(e0, w1a, w2a, dva)
    z2 = _light(e0, w1b, w2b, dvb)
    n1 = jnp.sqrt(jnp.sum(z1 * z1, axis=1, keepdims=True)) + 1e-12
    n2 = jnp.sqrt(jnp.sum(z2 * z2, axis=1, keepdims=True)) + 1e-12
    z1n = z1 / n1
    z2_s[...] = z2 / n2
    posd = jnp.sum(z1n * z2_s[...], axis=1) * (1.0 / TAU)

    def jstep(j, acc):
        tile = z2_s[pl.ds(j * 128, 128), :]
        t = lax.dot_general(z1n, tile, (((1,), (1,)), ((), ())),
                            preferred_element_type=jnp.float32)
        return acc + jnp.sum(jnp.exp(t * (1.0 / TAU)), axis=1)

    acc = lax.fori_loop(0, B // 128, jstep, jnp.zeros((B,), jnp.float32))
    out[0, 0] = jnp.mean(jnp.log(acc) - posd)


def _scalar_call(body, n_in, scratch=()):
    def run(*args):
        return pl.pallas_call(
            body,
            out_specs=pl.BlockSpec(memory_space=pltpu.SMEM),
            out_shape=jax.ShapeDtypeStruct((1, 1), jnp.float32),
            scratch_shapes=list(scratch),
        )(*args)
    return run


def kernel(user_table, item_table, g_idx, g_val, s1_idx, s1_val, s2_idx,
           s2_val, user_id, item_id, neg_item_id):
    del g_val, s1_val, s2_val
    e0 = jnp.concatenate([user_table, item_table], axis=0)

    wnd = RW * EM

    def pad_split(idx):
        h = idx.shape[1] // 2
        hp = ((h + wnd - 1) // wnd) * wnd
        pad = hp - h
        dstu = jnp.concatenate([idx[0, :h], jnp.full((pad,), HALF, jnp.int32)])
        dsti = jnp.concatenate([idx[0, h:], jnp.full((pad,), N, jnp.int32)])
        colu = jnp.concatenate([idx[1, :h], jnp.zeros((pad,), jnp.int32)])
        coli = jnp.concatenate([idx[1, h:], jnp.zeros((pad,), jnp.int32)])
        dst2 = jnp.concatenate([dstu, dsti]).reshape(-1, EM)
        col2 = jnp.concatenate([colu, coli]).reshape(-1, EM)
        return dst2, col2, hp

    dst_g, col_g, hp_g = pad_split(g_idx)
    dst_a, col_a, hp_s = pad_split(s1_idx)
    dst_b, col_b, _ = pad_split(s2_idx)

    deg_g, deg_a, deg_b = _make_deg(hp_g, hp_s)(dst_g, dst_a, dst_b)
    ivg, iva, ivb, qvg, qva, qvb = _tc_dinv(deg_g, deg_a, deg_b)

    y0g, y0a, y0b = _sc_scale(e0, ivg, iva, ivb)

    spmm1_g = _make_spmm(hp_g, scaled=True)
    spmm1_s = _make_spmm(hp_s, scaled=True)
    y1g = spmm1_g(dst_g, col_g, y0g, qvg)
    y1a = spmm1_s(dst_a, col_a, y0a, qva)
    y1b = spmm1_s(dst_b, col_b, y0b, qvb)

    spmm2_g = _make_spmm(hp_g, scaled=False)
    spmm2_s = _make_spmm(hp_s, scaled=False)
    u2g = spmm2_g(dst_g, col_g, y1g)
    u2a = spmm2_s(dst_a, col_a, y1a)
    u2b = spmm2_s(dst_b, col_b, y1b)

    (ue0, pe0, ne0, w1g_u, w1g_i, w1g_n, w1a_u, w1b_u, w1a_i, w1b_i,
     w2g_u, w2g_i, w2g_n, w2a_u, w2b_u, w2a_i, w2b_i,
     dv_g_u, dv_g_i, dv_g_n, dv_a_u, dv_b_u, dv_a_i, dv_b_i) = _sc_gather(
        user_id, item_id, neg_item_id, user_table, item_table,
        ivg, iva, ivb, y1g, y1a, y1b, u2g, u2a, u2b,
    )

    bpr_reg = _scalar_call(_bpr_body, 12)(
        ue0, pe0, ne0, w1g_u, w2g_u, dv_g_u, w1g_i, w2g_i, dv_g_i,
        w1g_n, w2g_n, dv_g_n)
    nce = _scalar_call(_nce_body, 7, scratch=[pltpu.VMEM((B, D), jnp.float32)])
    ssl_u = nce(ue0, w1a_u, w2a_u, dv_a_u, w1b_u, w2b_u, dv_b_u)
    ssl_i = nce(pe0, w1a_i, w2a_i, dv_a_i, w1b_i, w2b_i, dv_b_i)

    total = bpr_reg[0, 0] + (ssl_u[0, 0] + ssl_i[0, 0]) * LMBD_SSL
    return total


# speedup vs baseline: 8.1544x; 1.2439x over previous
import jax
import jax.numpy as jnp
from jax import lax
from jax.experimental import pallas as pl
from jax.experimental.pallas import tpu as pltpu
from jax.experimental.pallas import tpu_sc as plsc

U = 50000
N = 100000
D = 64
CW = 16
NQ = D // CW
B = 4096
TAU = 0.2
LMBD_REG = 1e-4
LMBD_SSL = 0.1

NC, NS, L = 2, 16, 16
NW = NC * NS
EM = 128
RW = 8
HALF = U
CHP = 50176
PT = 3136
PT_LAST = HALF - PT
NPAD = 102400

_mesh = plsc.VectorSubcoreMesh(core_axis_name="c", subcore_axis_name="s")
_sc_params = pltpu.CompilerParams(use_tc_tiling_on_sc=False)

_DRAIN_BLKS = ((0, 1024), (1024, 1024), (2048, 1024), (3072, PT - 3072))


def _zero_fill(ref, nvec):
    def body(i, _):
        ref[pl.ds(i * L, L)] = jnp.zeros((L,), jnp.float32)
        return 0
    lax.fori_loop(0, nvec, body, 0)


def _scale_rows(dst_v, sc_v, off, bs, fn):
    def gloop(g, _):
        chunk = sc_v[pl.ds(off + g * L, L)]
        for k in range(L):
            r = g * L + k
            dst_v[r, :] = fn(r, jnp.full((L,), chunk[k]))
        return 0
    lax.fori_loop(0, bs // L, gloop, 0)


def _make_deg(h_g, h_s):
    hr_g, hr_s = h_g // EM, h_s // EM

    def body(dg, ds1, ds2, og, os1, os2, idx_v, ones_v, zb_v, st_v, deg_sh,
             dsem):
        c = lax.axis_index("c")
        s = lax.axis_index("s")
        base = c * HALF
        startz = s * PT
        startd = jnp.minimum(s * PT, PT_LAST)

        def ob(i, _):
            ones_v[pl.ds(i * L, L)] = jnp.ones((L,), jnp.float32)
            return 0
        lax.fori_loop(0, EM // L, ob, 0)
        _zero_fill(zb_v, PT // L)

        for dst2d, out, hr in ((dg, og, hr_g), (ds1, os1, hr_s), (ds2, os2, hr_s)):
            pltpu.sync_copy(zb_v, deg_sh.at[pl.ds(startz, PT)])
            plsc.subcore_barrier()
            nw = hr // RW
            def wloop(j, _):
                w = j * NS + s
                @pl.when(w < nw)
                def _():
                    r0 = c * hr + w * RW
                    pltpu.sync_copy(dst2d.at[pl.ds(r0, RW)], idx_v)
                    for i in range(RW):
                        for k in range(EM // L):
                            idx_v[i, pl.ds(k * L, L)] = (
                                idx_v[i, pl.ds(k * L, L)] - base)
                    sps = [pltpu.async_copy(ones_v, deg_sh.at[idx_v.at[i]],
                                            dsem, add=True)
                           for i in range(RW)]
                    for sp in sps:
                        sp.wait()
                return 0
            lax.fori_loop(0, pl.cdiv(nw, NS), wloop, 0)
            plsc.subcore_barrier()
            pltpu.sync_copy(deg_sh.at[pl.ds(startd, PT)], st_v)
            pltpu.sync_copy(st_v, out.at[pl.ds(base + startd, PT)])
            plsc.subcore_barrier()

    shp = jax.ShapeDtypeStruct((N,), jnp.float32)
    return pl.kernel(
        body,
        out_type=(shp, shp, shp),
        mesh=_mesh,
        compiler_params=_sc_params,
        scratch_types=[
            pltpu.VMEM((RW, EM), jnp.int32),
            pltpu.VMEM((EM,), jnp.float32),
            pltpu.VMEM((PT,), jnp.float32),
            pltpu.VMEM((PT,), jnp.float32),
            pltpu.VMEM_SHARED((CHP,), jnp.float32),
            pltpu.SemaphoreType.DMA,
        ],
    )


def _tc_dinv(deg_g, deg_a, deg_b):
    def body(dg, da, db, ig, ia, ib, qg, qa, qb):
        for dref, iref, qref in ((dg, ig, qg), (da, ia, qa), (db, ib, qb)):
            m = jnp.maximum(dref[...], 1.0)
            dinv = lax.rsqrt(m)
            iref[...] = dinv
            qref[...] = dinv * dinv

    sp = jax.ShapeDtypeStruct((NPAD // 128, 128), jnp.float32)
    pad = lambda d: jnp.pad(d, (0, NPAD - N)).reshape(NPAD // 128, 128)
    outs = pl.pallas_call(
        body,
        out_shape=[sp] * 6,
    )(pad(deg_g), pad(deg_a), pad(deg_b))
    return [o.reshape(NPAD)[:N] for o in outs]


def _scale_body(e0, ivg, iva, ivb, og, oa, ob, e_v, o_v, dg_v, da_v, db_v):
    c = lax.axis_index("c")
    s = lax.axis_index("s")
    wid = s * NC + c
    start = jnp.minimum(wid * PT, N - PT)

    pltpu.sync_copy(ivg.at[pl.ds(start, PT)], dg_v)
    pltpu.sync_copy(iva.at[pl.ds(start, PT)], da_v)
    pltpu.sync_copy(ivb.at[pl.ds(start, PT)], db_v)

    for off, bs in _DRAIN_BLKS:
        pltpu.sync_copy(e0.at[pl.ds(start + off, bs)], e_v.at[pl.ds(0, bs)])
        for dv_v, out in ((dg_v, og), (da_v, oa), (db_v, ob)):
            def qloop(q, _):
                _scale_rows(o_v, dv_v, off, bs,
                            lambda r, sc: e_v[r, pl.ds(q * CW, CW)] * sc)
                pltpu.sync_copy(o_v.at[pl.ds(0, bs)],
                                out.at[pl.ds(q * N + start + off, bs)])
                return 0
            lax.fori_loop(0, NQ, qloop, 0)


def _sc_scale(e0, ivg, iva, ivb):
    sq = jax.ShapeDtypeStruct((NQ * N, CW), jnp.float32)
    return pl.kernel(
        _scale_body,
        out_type=(sq, sq, sq),
        mesh=_mesh,
        compiler_params=_sc_params,
        scratch_types=[
            pltpu.VMEM((1024, D), jnp.float32),
            pltpu.VMEM((1024, CW), jnp.float32),
            pltpu.VMEM((PT,), jnp.float32),
            pltpu.VMEM((PT,), jnp.float32),
            pltpu.VMEM((PT,), jnp.float32),
        ],
    )(e0, ivg, iva, ivb)


def _make_spmm(h, scaled):
    hr = h // EM

    def body(*refs):
        if scaled:
            (dst2d, col2d, y, scv, u, dst_a, col_a, cq_a, rows_a,
             dst_b, col_b, cq_b, rows_b, zb_v, st_v,
             sc_v, chunk_sh, semg, sems) = refs
        else:
            (dst2d, col2d, y, u, dst_a, col_a, cq_a, rows_a,
             dst_b, col_b, cq_b, rows_b, zb_v, st_v,
             sc_v, chunk_sh, semg, sems) = refs
            scv = None
        bufs_a = (dst_a, col_a, cq_a, rows_a)
        bufs_b = (dst_b, col_b, cq_b, rows_b)
        c = lax.axis_index("c")
        s = lax.axis_index("s")
        base = c * HALF
        startz = s * PT
        startd = jnp.minimum(s * PT, PT_LAST)
        nw = hr // RW

        def zb(i, _):
            zb_v[i, :] = jnp.zeros((L,), jnp.float32)
            return 0
        lax.fori_loop(0, 1024, zb, 0)
        if scaled:
            pltpu.sync_copy(scv.at[pl.ds(base + startd, PT)], sc_v)

        def qpass(q, _):
            qoff = q * N
            for zo, zs in _DRAIN_BLKS:
                pltpu.sync_copy(zb_v.at[pl.ds(0, zs)],
                                chunk_sh.at[pl.ds(startz + zo, zs)])
            plsc.subcore_barrier()

            def load_loc(w, bufs):
                dst_v, col_v, cq_v, _ = bufs
                r0 = c * hr + w * RW
                pltpu.sync_copy(dst2d.at[pl.ds(r0, RW)], dst_v)
                pltpu.sync_copy(col2d.at[pl.ds(r0, RW)], col_v)
                for i in range(RW):
                    for k in range(EM // L):
                        dst_v[i, pl.ds(k * L, L)] = (
                            dst_v[i, pl.ds(k * L, L)] - base)
                        cq_v[i, pl.ds(k * L, L)] = (
                            col_v[i, pl.ds(k * L, L)] + qoff)

            def fire_g(bufs):
                _, _, cq_v, rows_v = bufs
                return [pltpu.async_copy(y.at[cq_v.at[i]], rows_v.at[i], semg)
                        for i in range(RW)]

            def fire_s(bufs):
                dst_v, _, _, rows_v = bufs
                return [pltpu.async_copy(rows_v.at[i],
                                         chunk_sh.at[dst_v.at[i]], sems,
                                         add=True)
                        for i in range(RW)]

            def run_one(w, bufs):
                load_loc(w, bufs)
                for cp in fire_g(bufs):
                    cp.wait()
                for sp in fire_s(bufs):
                    sp.wait()

            jmax = pl.cdiv(nw, NS)

            def w2loop(j2, _):
                wa = (2 * j2) * NS + s
                wb = (2 * j2 + 1) * NS + s

                @pl.when(wb < nw)
                def _():
                    load_loc(wa, bufs_a)
                    ga = fire_g(bufs_a)
                    load_loc(wb, bufs_b)
                    for cp in ga:
                        cp.wait()
                    sa = fire_s(bufs_a)
                    gb = fire_g(bufs_b)
                    for sp in sa:
                        sp.wait()
                    for cp in gb:
                        cp.wait()
                    for sp in fire_s(bufs_b):
                        sp.wait()

                @pl.when((wa < nw) & (wb >= nw))
                def _():
                    run_one(wa, bufs_a)
                return 0
            lax.fori_loop(0, pl.cdiv(jmax, 2), w2loop, 0)
            plsc.subcore_barrier()

            for off, bs in _DRAIN_BLKS:
                pltpu.sync_copy(chunk_sh.at[pl.ds(startd + off, bs)],
                                st_v.at[pl.ds(0, bs)])
                if scaled:
                    _scale_rows(st_v, sc_v, off, bs,
                                lambda r, sc: st_v[r, :] * sc)
                pltpu.sync_copy(st_v.at[pl.ds(0, bs)],
                                u.at[pl.ds(qoff + base + startd + off, bs)])
            plsc.subcore_barrier()
            return 0
        lax.fori_loop(0, NQ, qpass, 0)

    sq = jax.ShapeDtypeStruct((NQ * N, CW), jnp.float32)
    return pl.kernel(
        body,
        out_type=sq,
        mesh=_mesh,
        compiler_params=_sc_params,
        scratch_types=[
            pltpu.VMEM((RW, EM), jnp.int32),
            pltpu.VMEM((RW, EM), jnp.int32),
            pltpu.VMEM((RW, EM), jnp.int32),
            pltpu.VMEM((RW, EM, CW), jnp.float32),
            pltpu.VMEM((RW, EM), jnp.int32),
            pltpu.VMEM((RW, EM), jnp.int32),
            pltpu.VMEM((RW, EM), jnp.int32),
            pltpu.VMEM((RW, EM, CW), jnp.float32),
            pltpu.VMEM((1024, CW), jnp.float32),
            pltpu.VMEM((1024, CW), jnp.float32),
            pltpu.VMEM((PT,), jnp.float32),
            pltpu.VMEM_SHARED((CHP, CW), jnp.float32),
            pltpu.SemaphoreType.DMA,
            pltpu.SemaphoreType.DMA,
        ],
    )


BW = B // NW


def _gather_body(uid, iid, nid, utab, itab, ivg, iva, ivb, y1g, y1a, y1b,
                 u2g, u2a, u2b,
                 ue0, pe0, ne0, w1g_u, w1g_i, w1g_n, w1a_u, w1b_u, w1a_i,
                 w1b_i, w2g_u, w2g_i, w2g_n, w2a_u, w2b_u, w2a_i, w2b_i,
                 dv_g_u, dv_g_i, dv_g_n, dv_a_u, dv_b_u, dv_a_i, dv_b_i,
                 uid_v, iid_v, nid_v, inode_v, nnode_v, qidx_v,
                 q_v, w_v, r64_v, dv_v, sem):
    c = lax.axis_index("c")
    s = lax.axis_index("s")
    wid = s * NC + c
    o = wid * BW

    pltpu.sync_copy(uid.at[pl.ds(o, BW)], uid_v)
    pltpu.sync_copy(iid.at[pl.ds(o, BW)], iid_v)
    pltpu.sync_copy(nid.at[pl.ds(o, BW)], nid_v)
    def off(i, _):
        inode_v[pl.ds(i * L, L)] = iid_v[pl.ds(i * L, L)] + U
        nnode_v[pl.ds(i * L, L)] = nid_v[pl.ds(i * L, L)] + U
        return 0
    lax.fori_loop(0, BW // L, off, 0)

    for tab, idx_v, out in ((utab, uid_v, ue0), (itab, iid_v, pe0),
                            (itab, nid_v, ne0)):
        pltpu.async_copy(tab.at[idx_v], r64_v, sem).wait()
        pltpu.sync_copy(r64_v, out.at[pl.ds(o, BW)])

    combos = ((uid_v, y1g, w1g_u), (inode_v, y1g, w1g_i), (nnode_v, y1g, w1g_n),
              (uid_v, y1a, w1a_u), (uid_v, y1b, w1b_u),
              (inode_v, y1a, w1a_i), (inode_v, y1b, w1b_i),
              (uid_v, u2g, w2g_u), (inode_v, u2g, w2g_i), (nnode_v, u2g, w2g_n),
              (uid_v, u2a, w2a_u), (uid_v, u2b, w2b_u),
              (inode_v, u2a, w2a_i), (inode_v, u2b, w2b_i))
    for idx_v, src, out in combos:
        for q in range(NQ):
            def qi(i, _):
                qidx_v[pl.ds(i * L, L)] = idx_v[pl.ds(i * L, L)] + q * N
                return 0
            lax.fori_loop(0, BW // L, qi, 0)
            pltpu.async_copy(src.at[qidx_v], q_v, sem).wait()
            def cpy(i, _):
                w_v[i, pl.ds(q * CW, CW)] = q_v[i, :]
                return 0
            lax.fori_loop(0, BW, cpy, 0)
        pltpu.sync_copy(w_v, out.at[pl.ds(o, BW)])

    for iv, idx_v, out in ((ivg, uid_v, dv_g_u), (ivg, inode_v, dv_g_i),
                           (ivg, nnode_v, dv_g_n), (iva, uid_v, dv_a_u),
                           (ivb, uid_v, dv_b_u), (iva, inode_v, dv_a_i),
                           (ivb, inode_v, dv_b_i)):
        pltpu.async_copy(iv.at[idx_v], dv_v, sem).wait()
        pltpu.sync_copy(dv_v, out.at[pl.ds(o, BW)])


def _sc_gather(*args):
    sr = jax.ShapeDtypeStruct((B, D), jnp.float32)
    sv = jax.ShapeDtypeStruct((B,), jnp.float32)
    return pl.kernel(
        _gather_body,
        out_type=(sr,) * 17 + (sv,) * 7,
        mesh=_mesh,
        compiler_params=_sc_params,
        scratch_types=[
            pltpu.VMEM((BW,), jnp.int32),
            pltpu.VMEM((BW,), jnp.int32),
            pltpu.VMEM((BW,), jnp.int32),
            pltpu.VMEM((BW,), jnp.int32),
            pltpu.VMEM((BW,), jnp.int32),
            pltpu.VMEM((BW,), jnp.int32),
            pltpu.VMEM((BW, CW), jnp.float32),
            pltpu.VMEM((BW, D), jnp.float32),
            pltpu.VMEM((BW, D), jnp.float32),
            pltpu.VMEM((BW,), jnp.float32),
            pltpu.SemaphoreType.DMA,
        ],
    )(*args)


def _light(e0r, w1, w2, dv):
    d = dv[...][:, None]
    return (e0r[...] + w1[...] / d + d * w2[...]) * jnp.float32(1.0 / 3.0)


def _bpr_body(ue0, pe0, ne0, w1u, w2u, dvu, w1i, w2i, dvi, w1n, w2n, dvn, out):
    ue = _light(ue0, w1u, w2u, dvu)
    pe = _light(pe0, w1i, w2i, dvi)
    ne = _light(ne0, w1n, w2n, dvn)
    x = jnp.sum(ue * pe, axis=1) - jnp.sum(ue * ne, axis=1)
    logsig = jnp.minimum(x, 0.0) - jnp.log(1.0 + jnp.exp(-jnp.abs(x)))
    bpr = -jnp.mean(logsig)
    reg = (LMBD_REG * 0.5 / B) * (jnp.sum(ue0[...] ** 2) + jnp.sum(pe0[...] ** 2)
                                  + jnp.sum(ne0[...] ** 2))
    out[0, 0] = bpr + reg * LMBD_REG


def _nce_body(e0, w1a, w2a, dva, w1b, w2b, dvb, out, z2_s):
    z1 = _light(e0, w1a, w2a, dva)
    z2 = _light(e0, w1b, w2b, dvb)
    n1 = jnp.sqrt(jnp.sum(z1 * z1, axis=1, keepdims=True)) + 1e-12
    n2 = jnp.sqrt(jnp.sum(z2 * z2, axis=1, keepdims=True)) + 1e-12
    z1n = z1 / n1
    z2_s[...] = z2 / n2
    posd = jnp.sum(z1n * z2_s[...], axis=1) * (1.0 / TAU)

    def jstep(j, acc):
        tile = z2_s[pl.ds(j * 128, 128), :]
        t = lax.dot_general(z1n, tile, (((1,), (1,)), ((), ())),
                            preferred_element_type=jnp.float32)
        return acc + jnp.sum(jnp.exp(t * (1.0 / TAU)), axis=1)

    acc = lax.fori_loop(0, B // 128, jstep, jnp.zeros((B,), jnp.float32))
    out[0, 0] = jnp.mean(jnp.log(acc) - posd)


def _scalar_call(body, n_in, scratch=()):
    def run(*args):
        return pl.pallas_call(
            body,
            out_specs=pl.BlockSpec(memory_space=pltpu.SMEM),
            out_shape=jax.ShapeDtypeStruct((1, 1), jnp.float32),
            scratch_shapes=list(scratch),
        )(*args)
    return run


def kernel(user_table, item_table, g_idx, g_val, s1_idx, s1_val, s2_idx,
           s2_val, user_id, item_id, neg_item_id):
    del g_val, s1_val, s2_val
    e0 = jnp.concatenate([user_table, item_table], axis=0)

    wnd = RW * EM

    def pad_split(idx):
        h = idx.shape[1] // 2
        hp = ((h + wnd - 1) // wnd) * wnd
        pad = hp - h
        dstu = jnp.concatenate([idx[0, :h], jnp.full((pad,), HALF, jnp.int32)])
        dsti = jnp.concatenate([idx[0, h:], jnp.full((pad,), N, jnp.int32)])
        colu = jnp.concatenate([idx[1, :h], jnp.zeros((pad,), jnp.int32)])
        coli = jnp.concatenate([idx[1, h:], jnp.zeros((pad,), jnp.int32)])
        dst2 = jnp.concatenate([dstu, dsti]).reshape(-1, EM)
        col2 = jnp.concatenate([colu, coli]).reshape(-1, EM)
        return dst2, col2, hp

    dst_g, col_g, hp_g = pad_split(g_idx)
    dst_a, col_a, hp_s = pad_split(s1_idx)
    dst_b, col_b, _ = pad_split(s2_idx)

    deg_g, deg_a, deg_b = _make_deg(hp_g, hp_s)(dst_g, dst_a, dst_b)
    ivg, iva, ivb, qvg, qva, qvb = _tc_dinv(deg_g, deg_a, deg_b)

    y0g, y0a, y0b = _sc_scale(e0, ivg, iva, ivb)

    spmm1_g = _make_spmm(hp_g, scaled=True)
    spmm1_s = _make_spmm(hp_s, scaled=True)
    y1g = spmm1_g(dst_g, col_g, y0g, qvg)
    y1a = spmm1_s(dst_a, col_a, y0a, qva)
    y1b = spmm1_s(dst_b, col_b, y0b, qvb)

    spmm2_g = _make_spmm(hp_g, scaled=False)
    spmm2_s = _make_spmm(hp_s, scaled=False)
    u2g = spmm2_g(dst_g, col_g, y1g)
    u2a = spmm2_s(dst_a, col_a, y1a)
    u2b = spmm2_s(dst_b, col_b, y1b)

    (ue0, pe0, ne0, w1g_u, w1g_i, w1g_n, w1a_u, w1b_u, w1a_i, w1b_i,
     w2g_u, w2g_i, w2g_n, w2a_u, w2b_u, w2a_i, w2b_i,
     dv_g_u, dv_g_i, dv_g_n, dv_a_u, dv_b_u, dv_a_i, dv_b_i) = _sc_gather(
        user_id, item_id, neg_item_id, user_table, item_table,
        ivg, iva, ivb, y1g, y1a, y1b, u2g, u2a, u2b,
    )

    bpr_reg = _scalar_call(_bpr_body, 12)(
        ue0, pe0, ne0, w1g_u, w2g_u, dv_g_u, w1g_i, w2g_i, dv_g_i,
        w1g_n, w2g_n, dv_g_n)
    nce = _scalar_call(_nce_body, 7, scratch=[pltpu.VMEM((B, D), jnp.float32)])
    ssl_u = nce(ue0, w1a_u, w2a_u, dv_a_u, w1b_u, w2b_u, dv_b_u)
    ssl_i = nce(pe0, w1a_i, w2a_i, dv_a_i, w1b_i, w2b_i, dv_b_i)

    total = bpr_reg[0, 0] + (ssl_u[0, 0] + ssl_i[0, 0]) * LMBD_SSL
    return total
